# baseline (speedup 1.0000x reference)
import functools

import jax
import jax.numpy as jnp
from jax import lax
from jax.experimental import pallas as pl
from jax.experimental.pallas import tpu as pltpu
from jax.experimental.pallas import tpu_sc as plsc

_D = 64
_B = 4096 * 200

_info = plsc.get_sparse_core_info()
_NC, _NS = _info.num_cores, _info.num_subcores
_NW = _NC * _NS
_BPW = _B // _NW
_CH = 512
_NCHUNK = _BPW // _CH

_mesh = plsc.VectorSubcoreMesh(core_axis_name="c", subcore_axis_name="s")


@functools.partial(
    pl.kernel,
    mesh=_mesh,
    out_type=jax.ShapeDtypeStruct((_B, _D), jnp.float32),
    scratch_types=[
        pltpu.VMEM((_BPW,), jnp.int32),
        pltpu.VMEM((_CH, _D), jnp.float32),
        pltpu.VMEM((_CH, _D), jnp.float32),
        pltpu.SemaphoreType.DMA,
        pltpu.SemaphoreType.DMA,
        pltpu.SemaphoreType.DMA,
        pltpu.SemaphoreType.DMA,
    ],
    compiler_params=pltpu.CompilerParams(use_tc_tiling_on_sc=False),
)
def _gather_kernel(idx_hbm, table_hbm, out_hbm, idx_v, buf0, buf1,
                   gsem0, gsem1, ssem0, ssem1):
    wid = lax.axis_index("s") * _NC + lax.axis_index("c")
    base = wid * _BPW
    pltpu.sync_copy(idx_hbm.at[pl.ds(base, _BPW)], idx_v)

    def start_gather(c, buf, sem):
        pltpu.async_copy(table_hbm.at[idx_v.at[pl.ds(c * _CH, _CH)]], buf, sem)

    def wait_gather(buf, sem):
        pltpu.make_async_copy(table_hbm.at[idx_v.at[pl.ds(0, _CH)]], buf,
                              sem).wait()

    def start_scatter(c, buf, sem):
        pltpu.async_copy(buf, out_hbm.at[pl.ds(base + c * _CH, _CH)], sem)

    def wait_scatter(buf, sem):
        pltpu.make_async_copy(buf, out_hbm.at[pl.ds(base, _CH)], sem).wait()

    start_gather(0, buf0, gsem0)

    def body(i, carry):
        c = i * 2
        wait_gather(buf0, gsem0)
        start_scatter(c, buf0, ssem0)

        @pl.when(c > 0)
        def _():
            wait_scatter(buf1, ssem1)

        start_gather(c + 1, buf1, gsem1)

        wait_gather(buf1, gsem1)
        start_scatter(c + 1, buf1, ssem1)
        wait_scatter(buf0, ssem0)

        @pl.when(c + 2 < _NCHUNK)
        def _():
            start_gather(c + 2, buf0, gsem0)

        return carry

    lax.fori_loop(0, _NCHUNK // 2, body, 0)
    wait_scatter(buf1, ssem1)


def kernel(input_tensors, word_embed):
    idx = input_tensors[0].reshape(-1)
    out = _gather_kernel(idx, word_embed)
    return out.reshape(4096, 200, _D)
